# Initial kernel scaffold; baseline (speedup 1.0000x reference)
import functools

import jax
import jax.numpy as jnp
from jax import lax
from jax.experimental import pallas as pl
from jax.experimental.pallas import tpu as pltpu
from jax.experimental.pallas import tpu_sc as plsc

_NC = 2
_NS = 16
_NT = _NC * _NS
_CB = 128


def _make_deg_kernel(NP, CH):
    mesh = plsc.VectorSubcoreMesh(core_axis_name="c", subcore_axis_name="s")
    zrows = NP // _NS

    @functools.partial(
        pl.kernel, mesh=mesh,
        out_type=jax.ShapeDtypeStruct((_NC, NP, 16), jnp.float32),
        scratch_types=[
            pltpu.VMEM((CH, _CB), jnp.int32),
            pltpu.VMEM((_CB, 16), jnp.float32),
            pltpu.VMEM_SHARED((NP, 16), jnp.float32),
        ],
    )
    def deg_kernel(dst_hbm, ones_hbm, zeros_hbm, out_hbm, dst_v, ones_v, acc):
        cid = lax.axis_index("c")
        sid = lax.axis_index("s")
        gwid = cid * _NS + sid
        pltpu.sync_copy(zeros_hbm, acc.at[pl.ds(sid * zrows, zrows)])
        pltpu.sync_copy(ones_hbm, ones_v)
        pltpu.sync_copy(dst_hbm.at[gwid], dst_v)
        plsc.subcore_barrier()

        @pl.loop(0, CH)
        def _(j):
            pltpu.sync_copy(ones_v, acc.at[dst_v.at[j]], add=True)

        plsc.subcore_barrier()
        pltpu.sync_copy(acc.at[pl.ds(sid * zrows, zrows)],
                        out_hbm.at[cid].at[pl.ds(sid * zrows, zrows)])

    return deg_kernel


def _make_agg_kernel(NP, CH, D):
    mesh = plsc.VectorSubcoreMesh(core_axis_name="c", subcore_axis_name="s")
    zrows = NP // _NS

    @functools.partial(
        pl.kernel, mesh=mesh,
        out_type=jax.ShapeDtypeStruct((_NC, NP, D), jnp.float32),
        scratch_types=[
            pltpu.VMEM((CH, _CB), jnp.int32),
            pltpu.VMEM((CH, _CB), jnp.int32),
            pltpu.VMEM((_CB, D), jnp.float32),
            pltpu.VMEM((_CB, D), jnp.float32),
            pltpu.VMEM_SHARED((NP, D), jnp.float32),
            pltpu.SemaphoreType.DMA,
            pltpu.SemaphoreType.DMA,
        ],
    )
    def agg_kernel(hs_hbm, src_hbm, dst_hbm, zeros_hbm, out_hbm,
                   src_v, dst_v, rows0, rows1, acc, sem0, sem1):
        cid = lax.axis_index("c")
        sid = lax.axis_index("s")
        gwid = cid * _NS + sid
        pltpu.sync_copy(zeros_hbm, acc.at[pl.ds(sid * zrows, zrows)])
        pltpu.sync_copy(src_hbm.at[gwid], src_v)
        pltpu.sync_copy(dst_hbm.at[gwid], dst_v)
        plsc.subcore_barrier()

        pltpu.async_copy(hs_hbm.at[src_v.at[0]], rows0, sem0)

        @pl.loop(0, (CH - 1) // 2)
        def _(i):
            j0 = 2 * i
            pltpu.async_copy(hs_hbm.at[src_v.at[j0 + 1]], rows1, sem1)
            pltpu.make_async_copy(hs_hbm.at[src_v.at[j0]], rows0, sem0).wait()
            pltpu.sync_copy(rows0, acc.at[dst_v.at[j0]], add=True)
            pltpu.async_copy(hs_hbm.at[src_v.at[j0 + 2]], rows0, sem0)
            pltpu.make_async_copy(hs_hbm.at[src_v.at[j0 + 1]], rows1, sem1).wait()
            pltpu.sync_copy(rows1, acc.at[dst_v.at[j0 + 1]], add=True)

        pltpu.make_async_copy(hs_hbm.at[src_v.at[CH - 1]], rows0, sem0).wait()
        pltpu.sync_copy(rows0, acc.at[dst_v.at[CH - 1]], add=True)

        plsc.subcore_barrier()
        pltpu.sync_copy(acc.at[pl.ds(sid * zrows, zrows)],
                        out_hbm.at[cid].at[pl.ds(sid * zrows, zrows)])

    return agg_kernel


def _mm1_body(x_ref, w_ref, h_ref):
    h_ref[...] = jnp.dot(x_ref[...], w_ref[...],
                         preferred_element_type=jnp.float32)


def _scale_body(c_ref, h_ref, hs_ref, dis_ref):
    c = c_ref[0, :, 0:1] + c_ref[1, :, 0:1]
    dis = lax.rsqrt(1.0 + c)
    h = h_ref[...]
    hs_ref[...] = h * dis
    dis_ref[...] = jnp.broadcast_to(dis, h.shape)


def _layer_body(agg_ref, hs_ref, dis_ref, b_ref, w_ref, out_ref):
    dis = dis_ref[...]
    s = agg_ref[0] + agg_ref[1] + hs_ref[...]
    t = jnp.maximum(s * dis + b_ref[...], 0.0)
    out_ref[...] = jnp.dot(t, w_ref[...],
                           preferred_element_type=jnp.float32) * dis


def _final_body(agg_ref, hs_ref, dis_ref, b_ref, wfc_ref, bfc_ref, out_ref):
    s = agg_ref[0] + agg_ref[1] + hs_ref[...]
    t = jnp.maximum(s * dis_ref[...] + b_ref[...], 0.0)
    out_ref[...] = jnp.dot(t, wfc_ref[...],
                           preferred_element_type=jnp.float32) + bfc_ref[...]


def _tc_call(body, NP, D, BN, n_out, in_specs):
    return pl.pallas_call(
        body,
        grid=(NP // BN,),
        in_specs=in_specs,
        out_specs=[pl.BlockSpec((BN, D), lambda i: (i, 0))
                   for _ in range(n_out)],
        out_shape=[jax.ShapeDtypeStruct((NP, D), jnp.float32)
                   for _ in range(n_out)],
    )


def kernel(x, edge_index, W1, b1, W2, b2, Wfc, bfc):
    N, D = x.shape
    E = edge_index.shape[1]
    NP = ((N + 2047) // 2048) * 2048
    CH = -(-E // (_NT * _CB))
    if CH % 2 == 0:
        CH += 1
    EP = _NT * CH * _CB
    BN = 1024

    src = edge_index[0]
    dst = edge_index[1]
    pad = EP - E
    src_p = jnp.concatenate([src, jnp.zeros((pad,), src.dtype)])
    dst_p = jnp.concatenate([dst, jnp.full((pad,), N, dst.dtype)])
    src3 = src_p.reshape(_NT, CH, _CB)
    dst3 = dst_p.reshape(_NT, CH, _CB)

    ones16 = jnp.ones((_CB, 16), jnp.float32)
    zeros16 = jnp.zeros((NP // _NS, 16), jnp.float32)
    zerosD = jnp.zeros((NP // _NS, D), jnp.float32)
    xp = jnp.pad(x, ((0, NP - N), (0, 0)))

    deg_k = _make_deg_kernel(NP, CH)
    agg_k = _make_agg_kernel(NP, CH, D)

    w_spec = pl.BlockSpec((D, D), lambda i: (0, 0))
    b_spec = pl.BlockSpec((1, D), lambda i: (0, 0))
    row_spec = pl.BlockSpec((BN, D), lambda i: (i, 0))
    agg_spec = pl.BlockSpec((_NC, BN, D), lambda i: (0, i, 0))
    cnt_spec = pl.BlockSpec((_NC, BN, 16), lambda i: (0, i, 0))

    counts = deg_k(dst3, ones16, zeros16)
    (h1,) = _tc_call(_mm1_body, NP, D, BN, 1,
                     [row_spec, w_spec])(xp, W1)
    hs1, dis = _tc_call(_scale_body, NP, D, BN, 2,
                        [cnt_spec, row_spec])(counts, h1)
    agg1 = agg_k(hs1, src3, dst3, zerosD)
    (hs2,) = _tc_call(_layer_body, NP, D, BN, 1,
                      [agg_spec, row_spec, row_spec, b_spec, w_spec])(
        agg1, hs1, dis, b1.reshape(1, D), W2)
    agg2 = agg_k(hs2, src3, dst3, zerosD)
    (out,) = _tc_call(_final_body, NP, D, BN, 1,
                      [agg_spec, row_spec, row_spec, b_spec, w_spec, b_spec])(
        agg2, hs2, dis, b2.reshape(1, D), Wfc, bfc.reshape(1, D))
    return out[:N]


# speedup vs baseline: 10.1529x; 10.1529x over previous
import dataclasses
import functools

import jax
import jax.numpy as jnp
from jax import lax
from jax.experimental import pallas as pl
from jax.experimental.pallas import tpu as pltpu
from jax.experimental.pallas import tpu_sc as plsc

_NC = 2
_NS = 16
_NT = _NC * _NS
_CB = 128
_NB = 2


def _make_deg_kernel(NP, IB):
    mesh = plsc.VectorSubcoreMesh(core_axis_name="c", subcore_axis_name="s")
    SEG = NP // _NS

    @functools.partial(
        pl.kernel, mesh=mesh,
        out_type=jax.ShapeDtypeStruct((_NC, NP), jnp.float32),
        scratch_types=[
            pltpu.VMEM((_NB, IB, _CB), jnp.int32),
            pltpu.VMEM((NP,), jnp.float32),
            pltpu.VMEM((SEG,), jnp.float32),
            pltpu.VMEM((SEG,), jnp.float32),
            pltpu.VMEM_SHARED((_NS, NP), jnp.float32),
        ],
        compiler_params=dataclasses.replace(pltpu.CompilerParams(),
                                            needs_layout_passes=False),
    )
    def deg_kernel(dst_hbm, out_hbm, dst_v, hist_v, tmp_v, red_v, shared):
        cid = lax.axis_index("c")
        sid = lax.axis_index("s")
        gwid = cid * _NS + sid
        zeros = jnp.zeros((16,), jnp.float32)
        ones = jnp.ones((16,), jnp.float32)

        @pl.loop(0, NP // 16)
        def _(i):
            hist_v[pl.ds(i * 16, 16)] = zeros

        pltpu.sync_copy(dst_hbm.at[gwid], dst_v)

        @pl.loop(0, _NB)
        def _(b):
            @pl.loop(0, IB)
            def _(j):
                @pl.loop(0, _CB // 16)
                def _(l):
                    idx = dst_v[b, j, pl.ds(l * 16, 16)]
                    plsc.addupdate_scatter(hist_v, [idx], ones)

        pltpu.sync_copy(hist_v, shared.at[sid])
        plsc.subcore_barrier()

        @pl.loop(0, SEG // 16)
        def _(i):
            red_v[pl.ds(i * 16, 16)] = zeros

        @pl.loop(0, _NS)
        def _(k):
            pltpu.sync_copy(shared.at[k].at[pl.ds(sid * SEG, SEG)], tmp_v)

            @pl.loop(0, SEG // 16)
            def _(i):
                sl = pl.ds(i * 16, 16)
                red_v[sl] = red_v[sl] + tmp_v[sl]

        pltpu.sync_copy(red_v, out_hbm.at[cid].at[pl.ds(sid * SEG, SEG)])

    return deg_kernel


def _make_agg_kernel(NP, IB, D):
    mesh = plsc.VectorSubcoreMesh(core_axis_name="c", subcore_axis_name="s")
    zrows = NP // _NS

    @functools.partial(
        pl.kernel, mesh=mesh,
        out_type=jax.ShapeDtypeStruct((_NC, NP, D), jnp.float32),
        scratch_types=[
            pltpu.VMEM((IB, _CB), jnp.int32),
            pltpu.VMEM((IB, _CB), jnp.int32),
            pltpu.VMEM((_CB, D), jnp.float32),
            pltpu.VMEM((_CB, D), jnp.float32),
            pltpu.VMEM_SHARED((NP, D), jnp.float32),
            pltpu.SemaphoreType.DMA,
            pltpu.SemaphoreType.DMA,
        ],
    )
    def agg_kernel(hs_hbm, src_hbm, dst_hbm, zeros_hbm, out_hbm,
                   src_v, dst_v, rows0, rows1, acc, sem0, sem1):
        cid = lax.axis_index("c")
        sid = lax.axis_index("s")
        gwid = cid * _NS + sid
        pltpu.sync_copy(zeros_hbm, acc.at[pl.ds(sid * zrows, zrows)])
        plsc.subcore_barrier()

        @pl.loop(0, _NB)
        def _(b):
            pltpu.sync_copy(src_hbm.at[gwid].at[b], src_v)
            pltpu.sync_copy(dst_hbm.at[gwid].at[b], dst_v)
            pltpu.async_copy(hs_hbm.at[src_v.at[0]], rows0, sem0)

            @pl.loop(0, IB // 2 - 1)
            def _(i):
                j0 = 2 * i
                pltpu.async_copy(hs_hbm.at[src_v.at[j0 + 1]], rows1, sem1)
                pltpu.make_async_copy(hs_hbm.at[src_v.at[j0]], rows0, sem0).wait()
                pltpu.sync_copy(rows0, acc.at[dst_v.at[j0]], add=True)
                pltpu.async_copy(hs_hbm.at[src_v.at[j0 + 2]], rows0, sem0)
                pltpu.make_async_copy(hs_hbm.at[src_v.at[j0 + 1]], rows1, sem1).wait()
                pltpu.sync_copy(rows1, acc.at[dst_v.at[j0 + 1]], add=True)

            pltpu.async_copy(hs_hbm.at[src_v.at[IB - 1]], rows1, sem1)
            pltpu.make_async_copy(hs_hbm.at[src_v.at[IB - 2]], rows0, sem0).wait()
            pltpu.sync_copy(rows0, acc.at[dst_v.at[IB - 2]], add=True)
            pltpu.make_async_copy(hs_hbm.at[src_v.at[IB - 1]], rows1, sem1).wait()
            pltpu.sync_copy(rows1, acc.at[dst_v.at[IB - 1]], add=True)

        plsc.subcore_barrier()
        pltpu.sync_copy(acc.at[pl.ds(sid * zrows, zrows)],
                        out_hbm.at[cid].at[pl.ds(sid * zrows, zrows)])

    return agg_kernel


def _mm1_body(x_ref, w_ref, h_ref):
    h_ref[...] = jnp.dot(x_ref[...], w_ref[...],
                         preferred_element_type=jnp.float32)


def _scale_body(c_ref, h_ref, hs_ref, dis_ref):
    c = c_ref[0] + c_ref[1]
    dis = lax.rsqrt(1.0 + c)
    h = h_ref[...]
    hs_ref[...] = h * dis
    dis_ref[...] = jnp.broadcast_to(dis, h.shape)


def _layer_body(agg_ref, hs_ref, dis_ref, b_ref, w_ref, out_ref):
    dis = dis_ref[...]
    s = agg_ref[0] + agg_ref[1] + hs_ref[...]
    t = jnp.maximum(s * dis + b_ref[...], 0.0)
    out_ref[...] = jnp.dot(t, w_ref[...],
                           preferred_element_type=jnp.float32) * dis


def _final_body(agg_ref, hs_ref, dis_ref, b_ref, wfc_ref, bfc_ref, out_ref):
    s = agg_ref[0] + agg_ref[1] + hs_ref[...]
    t = jnp.maximum(s * dis_ref[...] + b_ref[...], 0.0)
    out_ref[...] = jnp.dot(t, wfc_ref[...],
                           preferred_element_type=jnp.float32) + bfc_ref[...]


def _tc_call(body, NP, D, BN, n_out, in_specs):
    return pl.pallas_call(
        body,
        grid=(NP // BN,),
        in_specs=in_specs,
        out_specs=[pl.BlockSpec((BN, D), lambda i: (i, 0))
                   for _ in range(n_out)],
        out_shape=[jax.ShapeDtypeStruct((NP, D), jnp.float32)
                   for _ in range(n_out)],
    )


def kernel(x, edge_index, W1, b1, W2, b2, Wfc, bfc):
    N, D = x.shape
    E = edge_index.shape[1]
    NP = ((N + 2047) // 2048) * 2048
    CH = -(-E // (_NT * _CB))
    IB = 2 * (-(-CH // (2 * _NB)))
    EP = _NT * _NB * IB * _CB
    BN = 1024

    src = edge_index[0]
    dst = edge_index[1]
    pad = EP - E
    src_p = jnp.concatenate([src, jnp.zeros((pad,), src.dtype)])
    dst_p = jnp.concatenate([dst, jnp.full((pad,), N, dst.dtype)])
    src3 = src_p.reshape(_NT, _NB, IB, _CB)
    dst3 = dst_p.reshape(_NT, _NB, IB, _CB)

    zerosD = jnp.zeros((NP // _NS, D), jnp.float32)
    xp = jnp.pad(x, ((0, NP - N), (0, 0)))

    deg_k = _make_deg_kernel(NP, IB)
    agg_k = _make_agg_kernel(NP, IB, D)

    w_spec = pl.BlockSpec((D, D), lambda i: (0, 0))
    b_spec = pl.BlockSpec((1, D), lambda i: (0, 0))
    row_spec = pl.BlockSpec((BN, D), lambda i: (i, 0))
    agg_spec = pl.BlockSpec((_NC, BN, D), lambda i: (0, i, 0))
    cnt_spec = pl.BlockSpec((_NC, BN, 1), lambda i: (0, i, 0))

    counts = deg_k(dst3).reshape(_NC, NP, 1)
    (h1,) = _tc_call(_mm1_body, NP, D, BN, 1,
                     [row_spec, w_spec])(xp, W1)
    hs1, dis = _tc_call(_scale_body, NP, D, BN, 2,
                        [cnt_spec, row_spec])(counts, h1)
    agg1 = agg_k(hs1, src3, dst3, zerosD)
    (hs2,) = _tc_call(_layer_body, NP, D, BN, 1,
                      [agg_spec, row_spec, row_spec, b_spec, w_spec])(
        agg1, hs1, dis, b1.reshape(1, D), W2)
    agg2 = agg_k(hs2, src3, dst3, zerosD)
    (out,) = _tc_call(_final_body, NP, D, BN, 1,
                      [agg_spec, row_spec, row_spec, b_spec, w_spec, b_spec])(
        agg2, hs2, dis, b2.reshape(1, D), Wfc, bfc.reshape(1, D))
    return out[:N]


# speedup vs baseline: 10.4747x; 1.0317x over previous
import dataclasses
import functools

import jax
import jax.numpy as jnp
from jax import lax
from jax.experimental import pallas as pl
from jax.experimental.pallas import tpu as pltpu
from jax.experimental.pallas import tpu_sc as plsc

_NC = 2
_NS = 16
_NT = _NC * _NS
_CB = 128
_IB = 40


def _core_offsets(cid, sid, nb0, nb1):
    CH0, CH1 = nb0 * _IB, nb1 * _IB
    nb = jnp.where(cid == 0, nb0, nb1)
    start = jnp.where(cid == 0, sid * CH0, _NS * CH0 + sid * CH1)
    return start, nb


def _make_deg_kernel(NP, nb0, nb1):
    mesh = plsc.VectorSubcoreMesh(core_axis_name="c", subcore_axis_name="s")
    SEG = NP // _NS

    @functools.partial(
        pl.kernel, mesh=mesh,
        out_type=jax.ShapeDtypeStruct((_NC, NP), jnp.float32),
        scratch_types=[
            pltpu.VMEM((_IB, _CB), jnp.int32),
            pltpu.VMEM((NP,), jnp.float32),
            pltpu.VMEM((SEG,), jnp.float32),
            pltpu.VMEM((SEG,), jnp.float32),
            pltpu.VMEM_SHARED((_NS, NP), jnp.float32),
        ],
        compiler_params=dataclasses.replace(pltpu.CompilerParams(),
                                            needs_layout_passes=False),
    )
    def deg_kernel(dst_hbm, out_hbm, dst_v, hist_v, tmp_v, red_v, shared):
        cid = lax.axis_index("c")
        sid = lax.axis_index("s")
        start, nb = _core_offsets(cid, sid, nb0, nb1)
        zeros = jnp.zeros((16,), jnp.float32)
        ones = jnp.ones((16,), jnp.float32)

        @pl.loop(0, NP // 16)
        def _(i):
            hist_v[pl.ds(i * 16, 16)] = zeros

        @pl.loop(0, nb)
        def _(b):
            off = pl.multiple_of(start + b * _IB, 8)
            pltpu.sync_copy(dst_hbm.at[pl.ds(off, _IB)], dst_v)

            @pl.loop(0, _IB)
            def _(j):
                @pl.loop(0, _CB // 16)
                def _(l):
                    idx = dst_v[j, pl.ds(l * 16, 16)]
                    plsc.addupdate_scatter(hist_v, [idx], ones)

        pltpu.sync_copy(hist_v, shared.at[sid])
        plsc.subcore_barrier()

        @pl.loop(0, SEG // 16)
        def _(i):
            red_v[pl.ds(i * 16, 16)] = zeros

        @pl.loop(0, _NS)
        def _(k):
            pltpu.sync_copy(shared.at[k].at[pl.ds(sid * SEG, SEG)], tmp_v)

            @pl.loop(0, SEG // 16)
            def _(i):
                sl = pl.ds(i * 16, 16)
                red_v[sl] = red_v[sl] + tmp_v[sl]

        pltpu.sync_copy(red_v, out_hbm.at[cid].at[pl.ds(sid * SEG, SEG)])

    return deg_kernel


def _make_agg_kernel(NP, nb0, nb1, D):
    mesh = plsc.VectorSubcoreMesh(core_axis_name="c", subcore_axis_name="s")
    zrows = NP // _NS

    @functools.partial(
        pl.kernel, mesh=mesh,
        out_type=jax.ShapeDtypeStruct((_NC, NP, D), jnp.float32),
        scratch_types=[
            pltpu.VMEM((_IB, _CB), jnp.int32),
            pltpu.VMEM((_IB, _CB), jnp.int32),
            pltpu.VMEM((_CB, D), jnp.float32),
            pltpu.VMEM((_CB, D), jnp.float32),
            pltpu.VMEM_SHARED((NP, D), jnp.float32),
            pltpu.SemaphoreType.DMA,
            pltpu.SemaphoreType.DMA,
        ],
    )
    def agg_kernel(hs_hbm, src_hbm, dst_hbm, zeros_hbm, out_hbm,
                   src_v, dst_v, rows0, rows1, acc, sem0, sem1):
        cid = lax.axis_index("c")
        sid = lax.axis_index("s")
        start, nb = _core_offsets(cid, sid, nb0, nb1)
        pltpu.sync_copy(zeros_hbm, acc.at[pl.ds(sid * zrows, zrows)])
        plsc.subcore_barrier()

        @pl.loop(0, nb)
        def _(b):
            off = pl.multiple_of(start + b * _IB, 8)
            pltpu.sync_copy(src_hbm.at[pl.ds(off, _IB)], src_v)
            pltpu.sync_copy(dst_hbm.at[pl.ds(off, _IB)], dst_v)
            pltpu.async_copy(hs_hbm.at[src_v.at[0]], rows0, sem0)

            @pl.loop(0, _IB // 2 - 1)
            def _(i):
                j0 = 2 * i
                pltpu.async_copy(hs_hbm.at[src_v.at[j0 + 1]], rows1, sem1)
                pltpu.make_async_copy(hs_hbm.at[src_v.at[j0]], rows0, sem0).wait()
                pltpu.sync_copy(rows0, acc.at[dst_v.at[j0]], add=True)
                pltpu.async_copy(hs_hbm.at[src_v.at[j0 + 2]], rows0, sem0)
                pltpu.make_async_copy(hs_hbm.at[src_v.at[j0 + 1]], rows1, sem1).wait()
                pltpu.sync_copy(rows1, acc.at[dst_v.at[j0 + 1]], add=True)

            pltpu.async_copy(hs_hbm.at[src_v.at[_IB - 1]], rows1, sem1)
            pltpu.make_async_copy(hs_hbm.at[src_v.at[_IB - 2]], rows0, sem0).wait()
            pltpu.sync_copy(rows0, acc.at[dst_v.at[_IB - 2]], add=True)
            pltpu.make_async_copy(hs_hbm.at[src_v.at[_IB - 1]], rows1, sem1).wait()
            pltpu.sync_copy(rows1, acc.at[dst_v.at[_IB - 1]], add=True)

        plsc.subcore_barrier()
        pltpu.sync_copy(acc.at[pl.ds(sid * zrows, zrows)],
                        out_hbm.at[cid].at[pl.ds(sid * zrows, zrows)])

    return agg_kernel


def _mm1_body(x_ref, w_ref, h_ref):
    h_ref[...] = jnp.dot(x_ref[...], w_ref[...],
                         preferred_element_type=jnp.float32)


def _scale_body(c_ref, h_ref, hs_ref, dis_ref):
    c = c_ref[0] + c_ref[1]
    dis = lax.rsqrt(1.0 + c)
    h = h_ref[...]
    hs_ref[...] = h * dis
    dis_ref[...] = jnp.broadcast_to(dis, h.shape)


def _layer_body(agg_ref, hs_ref, dis_ref, b_ref, w_ref, out_ref):
    dis = dis_ref[...]
    s = agg_ref[0] + agg_ref[1] + hs_ref[...]
    t = jnp.maximum(s * dis + b_ref[...], 0.0)
    out_ref[...] = jnp.dot(t, w_ref[...],
                           preferred_element_type=jnp.float32) * dis


def _final_body(agg_ref, hs_ref, dis_ref, b_ref, wfc_ref, bfc_ref, out_ref):
    s = agg_ref[0] + agg_ref[1] + hs_ref[...]
    t = jnp.maximum(s * dis_ref[...] + b_ref[...], 0.0)
    out_ref[...] = jnp.dot(t, wfc_ref[...],
                           preferred_element_type=jnp.float32) + bfc_ref[...]


def _tc_call(body, NP, D, BN, n_out, in_specs):
    return pl.pallas_call(
        body,
        grid=(NP // BN,),
        in_specs=in_specs,
        out_specs=[pl.BlockSpec((BN, D), lambda i: (i, 0))
                   for _ in range(n_out)],
        out_shape=[jax.ShapeDtypeStruct((NP, D), jnp.float32)
                   for _ in range(n_out)],
    )


def kernel(x, edge_index, W1, b1, W2, b2, Wfc, bfc):
    N, D = x.shape
    E = edge_index.shape[1]
    NP = ((N + 2047) // 2048) * 2048
    TCH = -(-E // (_NS * _CB))
    nbt = -(-TCH // _IB)
    nb0 = max(1, min(nbt - 1, round(0.75 * nbt)))
    nb1 = nbt - nb0
    EP = _NS * nbt * _IB * _CB
    BN = 1024

    src = edge_index[0]
    dst = edge_index[1]
    pad = EP - E
    src_p = jnp.concatenate([src, jnp.zeros((pad,), src.dtype)])
    dst_p = jnp.concatenate([dst, jnp.full((pad,), N, dst.dtype)])
    src3 = src_p.reshape(_NS * nbt * _IB, _CB)
    dst3 = dst_p.reshape(_NS * nbt * _IB, _CB)

    zerosD = jnp.zeros((NP // _NS, D), jnp.float32)
    xp = jnp.pad(x, ((0, NP - N), (0, 0)))

    deg_k = _make_deg_kernel(NP, nb0, nb1)
    agg_k = _make_agg_kernel(NP, nb0, nb1, D)

    w_spec = pl.BlockSpec((D, D), lambda i: (0, 0))
    b_spec = pl.BlockSpec((1, D), lambda i: (0, 0))
    row_spec = pl.BlockSpec((BN, D), lambda i: (i, 0))
    agg_spec = pl.BlockSpec((_NC, BN, D), lambda i: (0, i, 0))
    cnt_spec = pl.BlockSpec((_NC, BN, 1), lambda i: (0, i, 0))

    counts = deg_k(dst3).reshape(_NC, NP, 1)
    (h1,) = _tc_call(_mm1_body, NP, D, BN, 1,
                     [row_spec, w_spec])(xp, W1)
    hs1, dis = _tc_call(_scale_body, NP, D, BN, 2,
                        [cnt_spec, row_spec])(counts, h1)
    agg1 = agg_k(hs1, src3, dst3, zerosD)
    (hs2,) = _tc_call(_layer_body, NP, D, BN, 1,
                      [agg_spec, row_spec, row_spec, b_spec, w_spec])(
        agg1, hs1, dis, b1.reshape(1, D), W2)
    agg2 = agg_k(hs2, src3, dst3, zerosD)
    (out,) = _tc_call(_final_body, NP, D, BN, 1,
                      [agg_spec, row_spec, row_spec, b_spec, w_spec, b_spec])(
        agg2, hs2, dis, b2.reshape(1, D), Wfc, bfc.reshape(1, D))
    return out[:N]
